# Initial kernel scaffold; baseline (speedup 1.0000x reference)
import jax
import jax.numpy as jnp
from jax.experimental import pallas as pl


def kernel(grid_nfeat, mesh_nfeat, edge_index, mesh2grid_efeat, We0, be0, We1, be1, ge, bge, Wf0, bf0, Wf1, bf1, gf, bgf, Wn0, bn0, Wn1, bn1, gn, bgn):
    raise NotImplementedError("write your pallas kernel here")


# speedup vs baseline: 3.1127x; 3.1127x over previous
import functools

import jax
import jax.numpy as jnp
from jax import lax
from jax.experimental import pallas as pl
from jax.experimental.pallas import tpu as pltpu
from jax.experimental.pallas import tpu_sc as plsc

F32 = jnp.float32

E = 600000
N_GRID = 100000
N_MESH = 10000
D = 128
DE = 4

NC = 2
NS = 16
NW = NC * NS

CHUNK = 128
NCHUNK = 148
PER_TILE = CHUNK * NCHUNK
EPAD = NW * PER_TILE

B_EDGE = 2048
B_NODE = 1000


def _ln(h, g, b):
    mu = jnp.mean(h, axis=-1, keepdims=True)
    var = jnp.mean((h - mu) ** 2, axis=-1, keepdims=True)
    return g * (h - mu) / jnp.sqrt(var + 1e-5) + b


def _dot(a, b):
    return jnp.dot(a, b, preferred_element_type=F32)


def _prep_body(mesh_ref, grid0_ref, wa_ref, wb_ref, mp_ref, gp_ref):
    mp_ref[...] = _dot(mesh_ref[...], wa_ref[...])
    gp_ref[...] = _dot(grid0_ref[...], wb_ref[...])


def _prep(mesh_nfeat, grid0, Wf0a, Wf0b):
    nblk = N_MESH // B_NODE
    return pl.pallas_call(
        _prep_body,
        grid=(nblk,),
        in_specs=[
            pl.BlockSpec((B_NODE, D), lambda i: (i, 0)),
            pl.BlockSpec((B_NODE, D), lambda i: (i, 0)),
            pl.BlockSpec((D, D), lambda i: (0, 0)),
            pl.BlockSpec((D, D), lambda i: (0, 0)),
        ],
        out_specs=[
            pl.BlockSpec((B_NODE, D), lambda i: (i, 0)),
            pl.BlockSpec((B_NODE, D), lambda i: (i, 0)),
        ],
        out_shape=[
            jax.ShapeDtypeStruct((N_MESH, D), F32),
            jax.ShapeDtypeStruct((N_MESH, D), F32),
        ],
    )(mesh_nfeat, grid0, Wf0a, Wf0b)


def _sc_gather_body(src_hbm, dst_hbm, mtab_hbm, gtab_hbm, out_hbm,
                    idx_s, idx_d, buf_m, buf_g, sem_m, sem_g):
    cid = lax.axis_index("c")
    sid = lax.axis_index("s")
    wid = sid * NC + cid
    base = wid * PER_TILE

    def chunk(g, carry):
        off = base + g * CHUNK
        pltpu.sync_copy(src_hbm.at[pl.ds(off, CHUNK)], idx_s)
        pltpu.sync_copy(dst_hbm.at[pl.ds(off, CHUNK)], idx_d)
        cm = pltpu.async_copy(mtab_hbm.at[idx_s], buf_m, sem_m)
        cg = pltpu.async_copy(gtab_hbm.at[idx_d], buf_g, sem_g)
        cm.wait()
        cg.wait()

        def add_row(i, c2):
            for j in range(D // 16):
                sl = pl.ds(j * 16, 16)
                buf_m[i, sl] = buf_m[i, sl] + buf_g[i, sl]
            return c2

        lax.fori_loop(0, CHUNK, add_row, 0)
        pltpu.sync_copy(buf_m, out_hbm.at[pl.ds(off, CHUNK)])
        return carry

    lax.fori_loop(0, NCHUNK, chunk, 0)


def _sc_gather(src_p, dst_p, mtab, gtab):
    mesh = plsc.VectorSubcoreMesh(
        core_axis_name="c", subcore_axis_name="s", num_cores=NC,
        num_subcores=NS)
    f = pl.kernel(
        _sc_gather_body,
        out_type=jax.ShapeDtypeStruct((EPAD, D), F32),
        mesh=mesh,
        scratch_types=[
            pltpu.VMEM((CHUNK,), jnp.int32),
            pltpu.VMEM((CHUNK,), jnp.int32),
            pltpu.VMEM((CHUNK, D), F32),
            pltpu.VMEM((CHUNK, D), F32),
            pltpu.SemaphoreType.DMA,
            pltpu.SemaphoreType.DMA,
        ],
    )
    return f(src_p, dst_p, mtab, gtab)


def _edge_body(ef_ref, gath_ref,
               We0_ref, be0_ref, We1_ref, be1_ref, ge_ref, bge_ref,
               Wf0c_ref, bf0_ref, Wf1_ref, bf1_ref, gf_ref, bgf_ref,
               out_ref):
    i = pl.program_id(0)
    u = jax.nn.silu(_dot(ef_ref[...], We0_ref[...]) + be0_ref[...])
    h = _dot(u, We1_ref[...]) + be1_ref[...]
    efeat = _ln(h, ge_ref[...], bge_ref[...])
    pre = _dot(efeat, Wf0c_ref[...]) + bf0_ref[...] + gath_ref[...]
    h2 = _dot(jax.nn.silu(pre), Wf1_ref[...]) + bf1_ref[...]
    e_upd = _ln(h2, gf_ref[...], bgf_ref[...])
    rows = i * B_EDGE + lax.broadcasted_iota(jnp.int32, (B_EDGE, 1), 0)
    out_ref[...] = jnp.where(rows < E, e_upd, 0.0)


def _edge(ef_p, gath, We0, be0, We1, be1, ge, bge, Wf0c, bf0, Wf1, bf1,
          gf, bgf):
    nblk = EPAD // B_EDGE
    full = lambda shape: pl.BlockSpec(shape, lambda i: (0, 0))
    return pl.pallas_call(
        _edge_body,
        grid=(nblk,),
        in_specs=[
            pl.BlockSpec((B_EDGE, DE), lambda i: (i, 0)),
            pl.BlockSpec((B_EDGE, D), lambda i: (i, 0)),
            full((DE, D)), full((1, D)), full((D, D)), full((1, D)),
            full((1, D)), full((1, D)),
            full((D, D)), full((1, D)), full((D, D)), full((1, D)),
            full((1, D)), full((1, D)),
        ],
        out_specs=pl.BlockSpec((B_EDGE, D), lambda i: (i, 0)),
        out_shape=jax.ShapeDtypeStruct((EPAD, D), F32),
    )(ef_p, gath, We0, be0, We1, be1, ge, bge, Wf0c, bf0, Wf1, bf1, gf, bgf)


def _sc_scatter_body(eupd_hbm, dst3_hbm, zeros_hbm, agg_hbm,
                     idx_t, buf_e, agg_s, sem_e):
    cid = lax.axis_index("c")
    sid = lax.axis_index("s")
    wid = sid * NC + cid
    base = wid * PER_TILE

    pltpu.sync_copy(dst3_hbm.at[wid], idx_t)

    @pl.when(sid == 0)
    def _():
        pltpu.sync_copy(zeros_hbm, agg_s)

    plsc.subcore_barrier()

    def chunk(g, carry):
        off = base + g * CHUNK
        pltpu.sync_copy(eupd_hbm.at[pl.ds(off, CHUNK)], buf_e)
        pltpu.sync_copy(buf_e, agg_s.at[idx_t.at[g]], add=True)
        return carry

    lax.fori_loop(0, NCHUNK, chunk, 0)
    plsc.subcore_barrier()

    @pl.when(sid == 0)
    def _():
        pltpu.sync_copy(agg_s, agg_hbm.at[cid])


def _sc_scatter(e_upd, dst3, zeros):
    mesh = plsc.VectorSubcoreMesh(
        core_axis_name="c", subcore_axis_name="s", num_cores=NC,
        num_subcores=NS)
    f = pl.kernel(
        _sc_scatter_body,
        out_type=jax.ShapeDtypeStruct((NC, N_MESH, D), F32),
        mesh=mesh,
        scratch_types=[
            pltpu.VMEM((NCHUNK, CHUNK), jnp.int32),
            pltpu.VMEM((CHUNK, D), F32),
            pltpu.VMEM_SHARED((N_MESH, D), F32),
            pltpu.SemaphoreType.DMA,
        ],
    )
    return f(e_upd, dst3, zeros)


def _node_a_body(grid_ref, agg0_ref, agg1_ref,
                 Wn0a_ref, Wn0b_ref, bn0_ref, Wn1_ref, bn1_ref,
                 gn_ref, bgn_ref, out_ref):
    g = grid_ref[...]
    agg = agg0_ref[...] + agg1_ref[...]
    pre = _dot(g, Wn0a_ref[...]) + _dot(agg, Wn0b_ref[...]) + bn0_ref[...]
    h = _dot(jax.nn.silu(pre), Wn1_ref[...]) + bn1_ref[...]
    out_ref[...] = g + _ln(h, gn_ref[...], bgn_ref[...])


def _node_b_body(grid_ref, Wn0a_ref, bn0_ref, Wn1_ref, bn1_ref,
                 gn_ref, bgn_ref, out_ref):
    g = grid_ref[...]
    pre = _dot(g, Wn0a_ref[...]) + bn0_ref[...]
    h = _dot(jax.nn.silu(pre), Wn1_ref[...]) + bn1_ref[...]
    out_ref[...] = g + _ln(h, gn_ref[...], bgn_ref[...])


def _node_a(grid0, agg0, agg1, Wn0a, Wn0b, bn0, Wn1, bn1, gn, bgn):
    nblk = N_MESH // B_NODE
    full = lambda shape: pl.BlockSpec(shape, lambda i: (0, 0))
    return pl.pallas_call(
        _node_a_body,
        grid=(nblk,),
        in_specs=[
            pl.BlockSpec((B_NODE, D), lambda i: (i, 0)),
            pl.BlockSpec((B_NODE, D), lambda i: (i, 0)),
            pl.BlockSpec((B_NODE, D), lambda i: (i, 0)),
            full((D, D)), full((D, D)), full((1, D)), full((D, D)),
            full((1, D)), full((1, D)), full((1, D)),
        ],
        out_specs=pl.BlockSpec((B_NODE, D), lambda i: (i, 0)),
        out_shape=jax.ShapeDtypeStruct((N_MESH, D), F32),
    )(grid0, agg0, agg1, Wn0a, Wn0b, bn0, Wn1, bn1, gn, bgn)


def _node_b(grid1, Wn0a, bn0, Wn1, bn1, gn, bgn):
    n = N_GRID - N_MESH
    nblk = n // B_NODE
    full = lambda shape: pl.BlockSpec(shape, lambda i: (0, 0))
    return pl.pallas_call(
        _node_b_body,
        grid=(nblk,),
        in_specs=[
            pl.BlockSpec((B_NODE, D), lambda i: (i, 0)),
            full((D, D)), full((1, D)), full((D, D)),
            full((1, D)), full((1, D)), full((1, D)),
        ],
        out_specs=pl.BlockSpec((B_NODE, D), lambda i: (i, 0)),
        out_shape=jax.ShapeDtypeStruct((n, D), F32),
    )(grid1, Wn0a, bn0, Wn1, bn1, gn, bgn)


def kernel(grid_nfeat, mesh_nfeat, edge_index, mesh2grid_efeat,
           We0, be0, We1, be1, ge, bge,
           Wf0, bf0, Wf1, bf1, gf, bgf,
           Wn0, bn0, Wn1, bn1, gn, bgn):
    src = edge_index[0].astype(jnp.int32)
    dst = edge_index[1].astype(jnp.int32)
    src_p = jnp.pad(src, (0, EPAD - E))
    dst_p = jnp.pad(dst, (0, EPAD - E))
    ef_p = jnp.pad(mesh2grid_efeat, ((0, EPAD - E), (0, 0)))

    Wf0a, Wf0b, Wf0c = Wf0[:D], Wf0[D:2 * D], Wf0[2 * D:]
    Wn0a, Wn0b = Wn0[:D], Wn0[D:]
    r = lambda v: v.reshape(1, D)
    grid0 = grid_nfeat[:N_MESH]
    grid1 = grid_nfeat[N_MESH:]

    mtab, gtab = _prep(mesh_nfeat, grid0, Wf0a, Wf0b)
    gath = _sc_gather(src_p, dst_p, mtab, gtab)
    e_upd = _edge(ef_p, gath, We0, r(be0), We1, r(be1), r(ge), r(bge),
                  Wf0c, r(bf0), Wf1, r(bf1), r(gf), r(bgf))
    dst3 = dst_p.reshape(NW, NCHUNK, CHUNK)
    zeros = jnp.zeros((N_MESH, D), F32)
    aggp = _sc_scatter(e_upd, dst3, zeros)
    out_a = _node_a(grid0, aggp[0], aggp[1], Wn0a, Wn0b, r(bn0),
                    Wn1, r(bn1), r(gn), r(bgn))
    out_b = _node_b(grid1, Wn0a, r(bn0), Wn1, r(bn1), r(gn), r(bgn))
    return jnp.concatenate([out_a, out_b], axis=0)
